# baseline (speedup 1.0000x reference)
import functools

import jax
import jax.numpy as jnp
from jax import lax
from jax.experimental import pallas as pl
from jax.experimental.pallas import tpu as pltpu
from jax.experimental.pallas import tpu_sc as plsc

N_FACTORS = 32
BATCH = 16384
N_ROWS = 1000000
PACK = 4
PACKED_W = N_FACTORS * PACK
PACKED_ROWS = N_ROWS // PACK
N_BLOCKS = N_ROWS // 128
TAIL = N_ROWS - N_BLOCKS * 128
NC = 2
NS = 16
L = 16
NW = NC * NS
BLOCKS_PER_W = -(-N_BLOCKS // NW)
B_PER_W = BATCH // NW
IDX_CHUNK = 128
N_CHUNKS = B_PER_W // IDX_CHUNK


def _transpose_block(fbuf, pbuf):
    f_lo = lax.iota(jnp.int32, L)
    f_hi = f_lo + L
    for u in range(128):
        uvec = jnp.full((L,), u, jnp.int32)
        v0 = plsc.load_gather(fbuf, [f_lo, uvec])
        v1 = plsc.load_gather(fbuf, [f_hi, uvec])
        p = u // PACK
        off = (u % PACK) * N_FACTORS
        pbuf[(p, pl.ds(off, L))] = v0
        pbuf[(p, pl.ds(off + L, L))] = v1


def _repack_body(ufT_hbm, ifT_hbm, utail_hbm, itail_hbm,
                 uout_hbm, iout_hbm,
                 fbuf0, fbuf1, pbuf0, pbuf1,
                 sem_in0, sem_in1, sem_out0, sem_out1, sem_tail):
    wid = lax.axis_index("s") * NC + lax.axis_index("c")
    base = wid * BLOCKS_PER_W

    def blk_col(k):
        b = lax.rem(base + k, N_BLOCKS)
        return pl.multiple_of(b * 128, 128), pl.multiple_of(b * 32, 8)

    for tab_hbm, out_hbm, fbufs, pbufs, sin, sout in (
        (ufT_hbm, uout_hbm, (fbuf0, fbuf1), (pbuf0, pbuf1),
         (sem_in0, sem_in1), (sem_out0, sem_out1)),
        (ifT_hbm, iout_hbm, (fbuf0, fbuf1), (pbuf0, pbuf1),
         (sem_in0, sem_in1), (sem_out0, sem_out1)),
    ):
        def fire_in(k, buf, sem, tab_hbm=tab_hbm):
            col, _ = blk_col(k)
            return pltpu.async_copy(
                tab_hbm.at[:, pl.ds(col, 128)], buf.at[pl.ds(0, N_FACTORS)],
                sem)

        def stage(k, buf, pb, sem_i, sem_o, first,
                  tab_hbm=tab_hbm, out_hbm=out_hbm):
            pltpu.make_async_copy(
                tab_hbm.at[:, pl.ds(0, 128)], buf.at[pl.ds(0, N_FACTORS)],
                sem_i).wait()

            @pl.when(jnp.logical_not(first))
            def _():
                pltpu.make_async_copy(
                    pb.at[...], out_hbm.at[pl.ds(0, 32)], sem_o).wait()

            _transpose_block(buf, pb)
            _, row = blk_col(k)
            return pltpu.async_copy(
                pb.at[...], out_hbm.at[pl.ds(row, 32)], sem_o)

        fire_in(0, fbufs[0], sin[0])
        fire_in(1, fbufs[1], sin[1])

        def super_body(i, carry):
            k0 = 2 * i
            stage(k0, fbufs[0], pbufs[0], sin[0], sout[0], i == 0)
            fire_in(k0 + 2, fbufs[0], sin[0])
            stage(k0 + 1, fbufs[1], pbufs[1], sin[1], sout[1], i == 0)

            @pl.when(k0 + 3 < BLOCKS_PER_W)
            def _():
                fire_in(k0 + 3, fbufs[1], sin[1])
            return carry

        lax.fori_loop(0, (BLOCKS_PER_W - 1) // 2, super_body, 0)
        stage(BLOCKS_PER_W - 1, fbufs[0], pbufs[0], sin[0], sout[0], False)
        pltpu.make_async_copy(
            pbufs[0].at[...], out_hbm.at[pl.ds(0, 32)], sout[0]).wait()
        pltpu.make_async_copy(
            pbufs[1].at[...], out_hbm.at[pl.ds(0, 32)], sout[1]).wait()

    @pl.when(wid == NW - 1)
    def _tail():
        for tail_hbm, out_hbm in ((utail_hbm, uout_hbm), (itail_hbm, iout_hbm)):
            pltpu.sync_copy(tail_hbm.at[...], fbuf0.at[...])
            for pp in range(TAIL // PACK):
                for a in range(PACK):
                    u = PACK * pp + a
                    for h in range(2):
                        src = (u, pl.ds(h * L, L))
                        dst = (pp, pl.ds(a * N_FACTORS + h * L, L))
                        pbuf0[dst] = fbuf0[src]
            cp = pltpu.async_copy(
                pbuf0.at[pl.ds(0, 16)],
                out_hbm.at[pl.ds(PACKED_ROWS - TAIL // PACK, 16)], sem_tail)
            cp.wait()


def _gather_body(user_hbm, item_hbm, uf_hbm, if_hbm, out_hbm,
                 uidx_v, iidx_v, ubase_v, ibase_v, ubuf, ibuf, out_v, sem):
    wid = lax.axis_index("s") * NC + lax.axis_index("c")
    base = wid * B_PER_W

    for j in range(N_CHUNKS):
        hsl = pl.ds(base + j * IDX_CHUNK, IDX_CHUNK)
        pltpu.sync_copy(user_hbm.at[hsl], uidx_v.at[j])
        pltpu.sync_copy(item_hbm.at[hsl], iidx_v.at[j])

    for j in range(N_CHUNKS):
        for i in range(IDX_CHUNK // L):
            sl = (j, pl.ds(i * L, L))
            u = uidx_v[sl] - 1
            ubase_v[sl] = (u & (PACK - 1)) * N_FACTORS
            uidx_v[sl] = lax.shift_right_logical(u, 2)
            it = iidx_v[sl] - 1
            ibase_v[sl] = (it & (PACK - 1)) * N_FACTORS
            iidx_v[sl] = lax.shift_right_logical(it, 2)

    lanes = lax.iota(jnp.int32, L)

    for j in range(N_CHUNKS):
        cu = pltpu.async_copy(uf_hbm.at[uidx_v.at[j]], ubuf.at[...], sem)
        ci = pltpu.async_copy(if_hbm.at[iidx_v.at[j]], ibuf.at[...], sem)
        cu.wait()
        ci.wait()

        def group(g, carry, j=j):
            gsl = pl.ds(g * L, L)
            rows = g * L + lanes
            ub = ubase_v[(j, gsl)]
            ib = ibase_v[(j, gsl)]
            acc = jnp.zeros((L,), jnp.float32)
            for f in range(N_FACTORS):
                uv = plsc.load_gather(ubuf, [rows, ub + f])
                iv = plsc.load_gather(ibuf, [rows, ib + f])
                acc = acc + uv * iv
            out_v[pl.ds(j * IDX_CHUNK + g * L, L)] = acc * 5.0
            return carry

        lax.fori_loop(0, IDX_CHUNK // L, group, 0)

    pltpu.sync_copy(out_v.at[...], out_hbm.at[pl.ds(base, B_PER_W)])


@jax.jit
def _mf_forward(user, item, user_factors, item_factors):
    mesh = plsc.VectorSubcoreMesh(core_axis_name="c", subcore_axis_name="s")

    ufT = user_factors.T
    ifT = item_factors.T
    utail = jnp.pad(user_factors[N_BLOCKS * 128:, :],
                    ((0, 128 - TAIL), (0, PACKED_W - N_FACTORS)))
    itail = jnp.pad(item_factors[N_BLOCKS * 128:, :],
                    ((0, 128 - TAIL), (0, PACKED_W - N_FACTORS)))

    repack = pl.kernel(
        _repack_body,
        mesh=mesh,
        out_type=(
            jax.ShapeDtypeStruct((PACKED_ROWS, PACKED_W), jnp.float32),
            jax.ShapeDtypeStruct((PACKED_ROWS, PACKED_W), jnp.float32),
        ),
        scratch_types=[
            pltpu.VMEM((128, 128), jnp.float32),
            pltpu.VMEM((128, 128), jnp.float32),
            pltpu.VMEM((32, PACKED_W), jnp.float32),
            pltpu.VMEM((32, PACKED_W), jnp.float32),
            pltpu.SemaphoreType.DMA,
            pltpu.SemaphoreType.DMA,
            pltpu.SemaphoreType.DMA,
            pltpu.SemaphoreType.DMA,
            pltpu.SemaphoreType.DMA,
        ],
        compiler_params=pltpu.CompilerParams(needs_layout_passes=False),
    )
    uf_packed, if_packed = repack(ufT, ifT, utail, itail)

    gather = pl.kernel(
        _gather_body,
        mesh=mesh,
        out_type=jax.ShapeDtypeStruct((BATCH,), jnp.float32),
        scratch_types=[
            pltpu.VMEM((N_CHUNKS, IDX_CHUNK), jnp.int32),
            pltpu.VMEM((N_CHUNKS, IDX_CHUNK), jnp.int32),
            pltpu.VMEM((N_CHUNKS, IDX_CHUNK), jnp.int32),
            pltpu.VMEM((N_CHUNKS, IDX_CHUNK), jnp.int32),
            pltpu.VMEM((IDX_CHUNK, PACKED_W), jnp.float32),
            pltpu.VMEM((IDX_CHUNK, PACKED_W), jnp.float32),
            pltpu.VMEM((B_PER_W,), jnp.float32),
            pltpu.SemaphoreType.DMA,
        ],
        compiler_params=pltpu.CompilerParams(needs_layout_passes=False),
    )
    return gather(user, item, uf_packed, if_packed)


def kernel(user, item, user_factors, item_factors):
    return _mf_forward(user, item, user_factors, item_factors)


# speedup vs baseline: 1.6272x; 1.6272x over previous
import functools

import jax
import jax.numpy as jnp
from jax import lax
from jax.experimental import pallas as pl
from jax.experimental.pallas import tpu as pltpu
from jax.experimental.pallas import tpu_sc as plsc

N_FACTORS = 32
BATCH = 16384
NC = 2
NS = 16
L = 16
NW = NC * NS
B_PER_W = BATCH // NW
IDX_CHUNK = 128
N_CHUNKS = B_PER_W // IDX_CHUNK


def _body(user_hbm, item_hbm, uf_hbm, if_hbm, out_hbm,
          uidx_v, iidx_v, urows_v, irows_v, out_v, sem):
    wid = lax.axis_index("s") * NC + lax.axis_index("c")
    base = wid * B_PER_W

    for j in range(N_CHUNKS):
        hsl = pl.ds(base + j * IDX_CHUNK, IDX_CHUNK)
        pltpu.sync_copy(user_hbm.at[hsl], uidx_v.at[j])
        pltpu.sync_copy(item_hbm.at[hsl], iidx_v.at[j])

    for j in range(N_CHUNKS):
        for i in range(IDX_CHUNK // L):
            sl = (j, pl.ds(i * L, L))
            uidx_v[sl] = uidx_v[sl] - 1
            iidx_v[sl] = iidx_v[sl] - 1

    copies = []
    for j in range(N_CHUNKS):
        rsl = pl.ds(j * IDX_CHUNK, IDX_CHUNK)
        copies.append(pltpu.async_copy(uf_hbm.at[uidx_v.at[j]],
                                       urows_v.at[rsl], sem))
        copies.append(pltpu.async_copy(if_hbm.at[iidx_v.at[j]],
                                       irows_v.at[rsl], sem))
    for c in copies:
        c.wait()

    lanes = lax.iota(jnp.int32, L)

    def group(g, carry):
        rows = g * L + lanes
        acc = jnp.zeros((L,), jnp.float32)
        for d in range(N_FACTORS):
            dcol = jnp.full((L,), d, jnp.int32)
            uv = plsc.load_gather(urows_v, [rows, dcol])
            iv = plsc.load_gather(irows_v, [rows, dcol])
            acc = acc + uv * iv
        out_v[pl.ds(g * L, L)] = acc * 5.0
        return carry

    lax.fori_loop(0, B_PER_W // L, group, 0)

    pltpu.sync_copy(out_v.at[...], out_hbm.at[pl.ds(base, B_PER_W)])


@jax.jit
def _mf_forward(user, item, user_factors, item_factors):
    mesh = plsc.VectorSubcoreMesh(core_axis_name="c", subcore_axis_name="s")
    f = pl.kernel(
        _body,
        mesh=mesh,
        out_type=jax.ShapeDtypeStruct((BATCH,), jnp.float32),
        scratch_types=[
            pltpu.VMEM((N_CHUNKS, IDX_CHUNK), jnp.int32),
            pltpu.VMEM((N_CHUNKS, IDX_CHUNK), jnp.int32),
            pltpu.VMEM((B_PER_W, N_FACTORS), jnp.float32),
            pltpu.VMEM((B_PER_W, N_FACTORS), jnp.float32),
            pltpu.VMEM((B_PER_W,), jnp.float32),
            pltpu.SemaphoreType.DMA,
        ],
        compiler_params=pltpu.CompilerParams(
            needs_layout_passes=False, use_tc_tiling_on_sc=False),
    )
    return f(user, item, user_factors, item_factors)


def kernel(user, item, user_factors, item_factors):
    return _mf_forward(user, item, user_factors, item_factors)
